# baseline (speedup 1.0000x reference)
import jax
import jax.numpy as jnp
from jax import lax
from jax.experimental import pallas as pl
from jax.experimental.pallas import tpu as pltpu
from jax.experimental.pallas import tpu_sc as plsc

VOCAB = 1000000
BINS = 100
D = 64
N = 4096 * 200
NC, NS = 2, 16
NW = NC * NS
PER_W = N // NW
G = 128
NG = PER_W // G
NB = 2


def _body(wids, sids, wtab, stab, out, widx_v, sidx_v, stab_v,
          rows0, rows1, obuf0, obuf1, gsem, ssem):
    rows = (rows0, rows1)
    obuf = (obuf0, obuf1)
    wid = lax.axis_index("s") * NC + lax.axis_index("c")
    pbase0 = wid * (PER_W // 2)
    pltpu.sync_copy(wids.at[wid], widx_v)
    pltpu.sync_copy(sids.at[wid], sidx_v)
    pltpu.sync_copy(stab, stab_v)

    def gather_desc(g, s):
        return pltpu.make_async_copy(
            wtab.at[widx_v.at[g]], rows[s], gsem.at[s])

    def scatter_desc(g, s):
        return pltpu.make_async_copy(
            obuf[s], out.at[pl.ds(pbase0 + g * (G // 2), G // 2)],
            ssem.at[s])

    for s in range(NB):
        gather_desc(s, s).start()

    @pl.loop(0, NG, step=NB)
    def _g0(g0):
        for s in range(NB):
            g = g0 + s
            gather_desc(g, s).wait()

            @pl.when(g >= NB)
            def _():
                scatter_desc(g - NB, s).wait()

            @pl.loop(0, G // 16, unroll=2)
            def _chunk(c):
                svec = sidx_v[g, pl.ds(c * 16, 16)]
                for k in range(16):
                    v = svec[k]
                    sid = v & 255
                    half = (v >> 8) * D
                    r = c * 16 + k
                    po = (c * 8 + k // 2, (k % 2) * D)
                    for q in range(D // 16):
                        obuf[s][po[0], pl.ds(po[1] + q * 16, 16)] = (
                            rows[s][r, pl.ds(half + q * 16, 16)]
                            + stab_v[sid, pl.ds(q * 16, 16)])

            scatter_desc(g, s).start()

            @pl.when(g + NB < NG)
            def _():
                gather_desc(g + NB, s).start()

    for g in (NG - NB, NG - 1):
        scatter_desc(g, g % NB).wait()


@jax.jit
def _sc_embed(wids, sids, wtab, stab):
    kern = pl.kernel(
        _body,
        out_type=jax.ShapeDtypeStruct((N // 2, 2 * D), jnp.float32),
        mesh=plsc.VectorSubcoreMesh(core_axis_name="c", subcore_axis_name="s"),
        compiler_params=pltpu.CompilerParams(needs_layout_passes=False),
        scratch_types=[
            pltpu.VMEM((NG, G), jnp.int32),
            pltpu.VMEM((NG, G), jnp.int32),
            pltpu.VMEM((BINS, D), jnp.float32),
            pltpu.VMEM((G, 2 * D), jnp.float32),
            pltpu.VMEM((G, 2 * D), jnp.float32),
            pltpu.VMEM((G // 2, 2 * D), jnp.float32),
            pltpu.VMEM((G // 2, 2 * D), jnp.float32),
            pltpu.SemaphoreType.DMA((NB,)),
            pltpu.SemaphoreType.DMA((NB,)),
        ],
    )
    return kern(wids, sids, wtab, stab)


def kernel(input_ids, scores_ids, words_emb, scores_emb):
    wflat = input_ids.reshape(NW, NG, G).astype(jnp.int32)
    sflat = scores_ids.reshape(NW, NG, G).astype(jnp.int32)
    wpair = wflat >> 1
    spack = sflat | ((wflat & 1) << 8)
    wtab2 = words_emb.reshape(VOCAB // 2, 2 * D)
    out = _sc_embed(wpair, spack, wtab2, scores_emb)
    return out.reshape(input_ids.shape + (D,))


# speedup vs baseline: 1.0141x; 1.0141x over previous
import jax
import jax.numpy as jnp
from jax import lax
from jax.experimental import pallas as pl
from jax.experimental.pallas import tpu as pltpu
from jax.experimental.pallas import tpu_sc as plsc

VOCAB = 1000000
BINS = 100
D = 64
N = 4096 * 200
NC, NS = 2, 16
NW = NC * NS
PER_W = N // NW
G = 128
NG = PER_W // G
NB = 4


def _body(wids, sids, wtab, stab, out, widx_v, sidx_v, stab_v,
          rows0, rows1, rows2, rows3, obuf0, obuf1, obuf2, obuf3,
          gsem, ssem):
    rows = (rows0, rows1, rows2, rows3)
    obuf = (obuf0, obuf1, obuf2, obuf3)
    wid = lax.axis_index("s") * NC + lax.axis_index("c")
    base0 = wid * PER_W
    pltpu.sync_copy(wids.at[wid], widx_v)
    pltpu.sync_copy(sids.at[wid], sidx_v)
    pltpu.sync_copy(stab, stab_v)

    def gather_desc(g, s):
        return pltpu.make_async_copy(
            wtab.at[widx_v.at[g]], rows[s], gsem.at[s])

    def scatter_desc(g, s):
        return pltpu.make_async_copy(
            obuf[s], out.at[pl.ds(base0 + g * G, G)], ssem.at[s])

    for s in range(2):
        gather_desc(s, s).start()

    @pl.loop(0, NG, step=NB)
    def _g0(g0):
        for s in range(NB):
            g = g0 + s
            gather_desc(g, s).wait()

            @pl.loop(0, G // 16, unroll=4)
            def _chunk(c):
                svec = sidx_v[g, pl.ds(c * 16, 16)]
                r0 = c * 16
                for k in range(16):
                    sid = svec[k]
                    for q in range(D // 16):
                        sl = pl.ds(q * 16, 16)
                        obuf[s][r0 + k, sl] = (
                            rows[s][r0 + k, sl] + stab_v[sid, sl])

            scatter_desc(g, s).start()

            sp = (s + 2) % NB

            @pl.when(g >= 2)
            def _():
                scatter_desc(g - 2, sp).wait()

            @pl.when(g + 2 < NG)
            def _():
                gather_desc(g + 2, sp).start()

    for g in (NG - 2, NG - 1):
        scatter_desc(g, g % NB).wait()


@jax.jit
def _sc_embed(wids, sids, wtab, stab):
    kern = pl.kernel(
        _body,
        out_type=jax.ShapeDtypeStruct((N, D), jnp.float32),
        mesh=plsc.VectorSubcoreMesh(core_axis_name="c", subcore_axis_name="s"),
        compiler_params=pltpu.CompilerParams(use_tc_tiling_on_sc=False,
                                             needs_layout_passes=False),
        scratch_types=[
            pltpu.VMEM((NG, G), jnp.int32),
            pltpu.VMEM((NG, G), jnp.int32),
            pltpu.VMEM((BINS, D), jnp.float32),
            pltpu.VMEM((G, D), jnp.float32),
            pltpu.VMEM((G, D), jnp.float32),
            pltpu.VMEM((G, D), jnp.float32),
            pltpu.VMEM((G, D), jnp.float32),
            pltpu.VMEM((G, D), jnp.float32),
            pltpu.VMEM((G, D), jnp.float32),
            pltpu.VMEM((G, D), jnp.float32),
            pltpu.VMEM((G, D), jnp.float32),
            pltpu.SemaphoreType.DMA((NB,)),
            pltpu.SemaphoreType.DMA((NB,)),
        ],
    )
    return kern(wids, sids, wtab, stab)


def kernel(input_ids, scores_ids, words_emb, scores_emb):
    wids = input_ids.reshape(NW, NG, G).astype(jnp.int32)
    sids = scores_ids.reshape(NW, NG, G).astype(jnp.int32)
    out = _sc_embed(wids, sids, words_emb, scores_emb)
    return out.reshape(input_ids.shape + (D,))


# speedup vs baseline: 1.0318x; 1.0174x over previous
import jax
import jax.numpy as jnp
from jax import lax
from jax.experimental import pallas as pl
from jax.experimental.pallas import tpu as pltpu
from jax.experimental.pallas import tpu_sc as plsc

VOCAB = 1000000
BINS = 100
D = 64
N = 4096 * 200
NC, NS = 2, 16
NW = NC * NS
PER_W = N // NW
G = 128
NG = PER_W // G
NB = 4


def _body(wids, sids, wtab, stab, out, widx_v, sidx_v, stab_v,
          rows0, rows1, rows2, rows3, obuf0, obuf1, obuf2, obuf3,
          gsem, ssem):
    rows = (rows0, rows1, rows2, rows3)
    obuf = (obuf0, obuf1, obuf2, obuf3)
    wid = lax.axis_index("s") * NC + lax.axis_index("c")
    base0 = wid * PER_W
    pltpu.sync_copy(wids.at[wid], widx_v)
    pltpu.sync_copy(sids.at[wid], sidx_v)
    pltpu.sync_copy(stab, stab_v)

    def gather_desc(g, s):
        return pltpu.make_async_copy(
            wtab.at[widx_v.at[g]], rows[s], gsem.at[s])

    def scatter_desc(g, s):
        return pltpu.make_async_copy(
            obuf[s], out.at[pl.ds(base0 + g * G, G)], ssem.at[s])

    for s in range(2):
        gather_desc(s, s).start()

    @pl.loop(0, NG, step=NB)
    def _g0(g0):
        for s in range(NB):
            g = g0 + s
            gather_desc(g, s).wait()

            @pl.loop(0, G // 16, unroll=2)
            def _chunk(c):
                svec = sidx_v[g, pl.ds(c * 16, 16)]
                r0 = c * 16
                for k in range(16):
                    sid = svec[k]
                    for q in range(D // 16):
                        sl = pl.ds(q * 16, 16)
                        obuf[s][r0 + k, sl] = (
                            rows[s][r0 + k, sl] + stab_v[sid, sl])

            scatter_desc(g, s).start()

            sp = (s + 2) % NB

            @pl.when(g >= 2)
            def _():
                scatter_desc(g - 2, sp).wait()

            @pl.when(g + 2 < NG)
            def _():
                gather_desc(g + 2, sp).start()

    for g in (NG - 2, NG - 1):
        scatter_desc(g, g % NB).wait()


@jax.jit
def _sc_embed(wids, sids, wtab, stab):
    kern = pl.kernel(
        _body,
        out_type=jax.ShapeDtypeStruct((N, D), jnp.float32),
        mesh=plsc.VectorSubcoreMesh(core_axis_name="c", subcore_axis_name="s"),
        compiler_params=pltpu.CompilerParams(use_tc_tiling_on_sc=False,
                                             needs_layout_passes=False),
        scratch_types=[
            pltpu.VMEM((NG, G), jnp.int32),
            pltpu.VMEM((NG, G), jnp.int32),
            pltpu.VMEM((BINS, D), jnp.float32),
            pltpu.VMEM((G, D), jnp.float32),
            pltpu.VMEM((G, D), jnp.float32),
            pltpu.VMEM((G, D), jnp.float32),
            pltpu.VMEM((G, D), jnp.float32),
            pltpu.VMEM((G, D), jnp.float32),
            pltpu.VMEM((G, D), jnp.float32),
            pltpu.VMEM((G, D), jnp.float32),
            pltpu.VMEM((G, D), jnp.float32),
            pltpu.SemaphoreType.DMA((NB,)),
            pltpu.SemaphoreType.DMA((NB,)),
        ],
    )
    return kern(wids, sids, wtab, stab)


def kernel(input_ids, scores_ids, words_emb, scores_emb):
    wids = input_ids.reshape(NW, NG, G).astype(jnp.int32)
    sids = scores_ids.reshape(NW, NG, G).astype(jnp.int32)
    out = _sc_embed(wids, sids, words_emb, scores_emb)
    return out.reshape(input_ids.shape + (D,))
